# Initial kernel scaffold; baseline (speedup 1.0000x reference)
import jax
import jax.numpy as jnp
from jax import lax
from jax.experimental import pallas as pl
from jax.experimental.pallas import tpu as pltpu
from jax.experimental.pallas import tpu_sc as plsc

T, B = 200, 16384
VOCAB, DIM = 115, 6
N = T * B
NC, NS, L = 2, 16, 16
NW = NC * NS
PER_W = N // NW
CHUNK = 2048
NCHUNK = PER_W // CHUNK
GROUPS = CHUNK // L
TABLE_PAD = 128


def _sc_kernel(table_hbm, idx_hbm, out_hbm, table_v, idx_v, out_v):
    wid = lax.axis_index("s") * NC + lax.axis_index("c")
    base = wid * PER_W

    pltpu.sync_copy(table_hbm, table_v)

    lane6 = jax.lax.iota(jnp.int32, L) * 6

    def chunk_body(c, _):
        cbase = base + c * CHUNK
        pltpu.sync_copy(idx_hbm.at[pl.ds(cbase, CHUNK)], idx_v)

        def group_body(g, _):
            tv = idx_v[pl.ds(g * L, L)]
            tv6 = tv * 6
            pos = lane6 + g * (L * 6)
            for d in range(DIM):
                vals = plsc.load_gather(table_v, [tv6 + d])
                plsc.store_scatter(out_v, [pos + d], vals)
            return 0

        lax.fori_loop(0, GROUPS, group_body, 0, unroll=2)
        pltpu.sync_copy(out_v, out_hbm.at[pl.ds(cbase * 6, CHUNK * 6)])
        return 0

    lax.fori_loop(0, NCHUNK, chunk_body, 0)


@jax.jit
def kernel(inputs, W):
    idx_flat = inputs.reshape(-1).astype(jnp.int32)
    table_flat = jnp.pad(W, ((0, TABLE_PAD - VOCAB), (0, 0))).reshape(-1)

    mesh = plsc.VectorSubcoreMesh(core_axis_name="c", subcore_axis_name="s")
    out_flat = pl.kernel(
        _sc_kernel,
        out_type=jax.ShapeDtypeStruct((N * DIM,), jnp.float32),
        mesh=mesh,
        scratch_types=[
            pltpu.VMEM((TABLE_PAD * DIM,), jnp.float32),
            pltpu.VMEM((CHUNK,), jnp.int32),
            pltpu.VMEM((CHUNK * DIM,), jnp.float32),
        ],
    )(table_flat, idx_flat)
    return out_flat.reshape(N, DIM)


# speedup vs baseline: 5.6741x; 5.6741x over previous
import jax
import jax.numpy as jnp
from jax import lax
from jax.experimental import pallas as pl
from jax.experimental.pallas import tpu as pltpu
from jax.experimental.pallas import tpu_sc as plsc

T, B = 200, 16384
VOCAB, DIM = 115, 6
N = T * B
NC, NS, L = 2, 16, 16
NW = NC * NS
PER_W = N // NW
CHUNK = 2048
NCHUNK = PER_W // CHUNK
GROUPS = CHUNK // L
TABLE_PAD = 128


def _sc_kernel(table_hbm, idx_hbm, out_hbm, table_v, idx_v, out_v):
    wid = lax.axis_index("s") * NC + lax.axis_index("c")
    base = wid * PER_W

    pltpu.sync_copy(table_hbm, table_v)

    lane6 = jax.lax.iota(jnp.int32, L) * 6

    def chunk_body(c, _):
        cbase = base + c * CHUNK
        pltpu.sync_copy(idx_hbm.at[pl.ds(cbase, CHUNK)], idx_v)

        def group_body(g, _):
            tv = idx_v[pl.ds(g * L, L)]
            tv6 = tv * 6
            pos = lane6 + g * (L * 6)
            for d in range(DIM):
                vals = plsc.load_gather(table_v, [tv6 + d])
                plsc.store_scatter(out_v, [pos + d], vals)
            return 0

        lax.fori_loop(0, GROUPS, group_body, 0, unroll=2)
        pltpu.sync_copy(out_v, out_hbm.at[pl.ds(cbase * 6, CHUNK * 6)])
        return 0

    lax.fori_loop(0, NCHUNK, chunk_body, 0)


@jax.jit
def kernel(inputs, W):
    idx_flat = inputs.reshape(-1).astype(jnp.int32)
    table_flat = jnp.pad(W, ((0, TABLE_PAD - VOCAB), (0, 0))).reshape(-1)

    mesh = plsc.VectorSubcoreMesh(core_axis_name="c", subcore_axis_name="s")
    out_flat = pl.kernel(
        _sc_kernel,
        out_type=jax.ShapeDtypeStruct((N * DIM,), jnp.float32),
        mesh=mesh,
        compiler_params=pltpu.CompilerParams(
            needs_layout_passes=False,
            use_tc_tiling_on_sc=False,
        ),
        scratch_types=[
            pltpu.VMEM((TABLE_PAD * DIM,), jnp.float32),
            pltpu.VMEM((CHUNK,), jnp.int32),
            pltpu.VMEM((CHUNK * DIM,), jnp.float32),
        ],
    )(table_flat, idx_flat)
    return out_flat.reshape(N, DIM)


# speedup vs baseline: 6.2526x; 1.1020x over previous
import jax
import jax.numpy as jnp
from jax import lax
from jax.experimental import pallas as pl
from jax.experimental.pallas import tpu as pltpu
from jax.experimental.pallas import tpu_sc as plsc

T, B = 200, 16384
VOCAB, DIM = 115, 6
N = T * B
NC, NS, L = 2, 16, 16
NW = NC * NS
PER_W = N // NW
CHUNK = 2048
NCHUNK = PER_W // CHUNK
GROUPS = CHUNK // L
TABLE_PAD = 128


def _sc_kernel(table_hbm, idx_hbm, out_hbm,
               table_v, idx0, idx1, out0, out1,
               sem_i0, sem_i1, sem_o0, sem_o1):
    wid = lax.axis_index("s") * NC + lax.axis_index("c")
    base = wid * PER_W

    pltpu.sync_copy(table_hbm, table_v)

    lane6 = jax.lax.iota(jnp.int32, L) * 6

    def start_idx(c, buf, sem):
        pltpu.async_copy(idx_hbm.at[pl.ds(base + c * CHUNK, CHUNK)], buf, sem)

    def wait_idx(c, buf, sem):
        pltpu.make_async_copy(
            idx_hbm.at[pl.ds(base + c * CHUNK, CHUNK)], buf, sem).wait()

    def start_out(c, buf, sem):
        pltpu.async_copy(
            buf, out_hbm.at[pl.ds((base + c * CHUNK) * 6, CHUNK * 6)], sem)

    def wait_out(c, buf, sem):
        pltpu.make_async_copy(
            buf, out_hbm.at[pl.ds((base + c * CHUNK) * 6, CHUNK * 6)],
            sem).wait()

    def compute(ibuf, obuf):
        @plsc.parallel_loop(0, GROUPS, unroll=8)
        def _(g):
            tv = ibuf[pl.ds(g * L, L)]
            tv6 = tv * 6
            pos = lane6 + g * (L * 6)
            for d in range(DIM):
                vals = plsc.load_gather(table_v, [tv6 + d])
                plsc.store_scatter(obuf, [pos + d], vals)

    start_idx(0, idx0, sem_i0)

    def pair_body(p, _):
        c0 = p * 2
        c1 = c0 + 1
        start_idx(c1, idx1, sem_i1)
        wait_idx(c0, idx0, sem_i0)

        @pl.when(p > 0)
        def _():
            wait_out(c0 - 2, out0, sem_o0)
        compute(idx0, out0)
        start_out(c0, out0, sem_o0)

        @pl.when(p < NCHUNK // 2 - 1)
        def _():
            start_idx(c0 + 2, idx0, sem_i0)
        wait_idx(c1, idx1, sem_i1)

        @pl.when(p > 0)
        def _():
            wait_out(c1 - 2, out1, sem_o1)
        compute(idx1, out1)
        start_out(c1, out1, sem_o1)
        return 0

    lax.fori_loop(0, NCHUNK // 2, pair_body, 0)
    wait_out(NCHUNK - 2, out0, sem_o0)
    wait_out(NCHUNK - 1, out1, sem_o1)


@jax.jit
def kernel(inputs, W):
    idx_flat = inputs.reshape(-1).astype(jnp.int32)
    table_flat = jnp.pad(W, ((0, TABLE_PAD - VOCAB), (0, 0))).reshape(-1)

    mesh = plsc.VectorSubcoreMesh(core_axis_name="c", subcore_axis_name="s")
    out_flat = pl.kernel(
        _sc_kernel,
        out_type=jax.ShapeDtypeStruct((N * DIM,), jnp.float32),
        mesh=mesh,
        compiler_params=pltpu.CompilerParams(
            needs_layout_passes=False,
            use_tc_tiling_on_sc=False,
        ),
        scratch_types=[
            pltpu.VMEM((TABLE_PAD * DIM,), jnp.float32),
            pltpu.VMEM((CHUNK,), jnp.int32),
            pltpu.VMEM((CHUNK,), jnp.int32),
            pltpu.VMEM((CHUNK * DIM,), jnp.float32),
            pltpu.VMEM((CHUNK * DIM,), jnp.float32),
            pltpu.SemaphoreType.DMA,
            pltpu.SemaphoreType.DMA,
            pltpu.SemaphoreType.DMA,
            pltpu.SemaphoreType.DMA,
        ],
    )(table_flat, idx_flat)
    return out_flat.reshape(N, DIM)


# speedup vs baseline: 165.9434x; 26.5397x over previous
import jax
import jax.numpy as jnp
from jax import lax
from jax.experimental import pallas as pl
from jax.experimental.pallas import tpu as pltpu
from jax.experimental.pallas import tpu_sc as plsc

T, B = 200, 16384
VOCAB, DIM = 115, 6
N = T * B
NC, NS, L = 2, 16, 16
NW = NC * NS
PER_W = N // NW
TILE = 1024
NTILES = N // 128
TPW = NTILES // NW
TPC = 16
CHUNK = TPC * 128
NCHUNK = TPW // TPC
GROUPS = CHUNK // L
OUT_CH = TPC * TILE


def _sc_kernel(table_hbm, idx_hbm, out_hbm,
               table_v, idx0, idx1, out0, out1,
               sem_i0, sem_i1, sem_o0, sem_o1):
    wid = lax.axis_index("s") * NC + lax.axis_index("c")
    ibase = wid * PER_W
    obase = wid * (TPW * TILE)

    pltpu.sync_copy(table_hbm, table_v)

    zeros = jnp.zeros((L,), jnp.float32)
    for obuf in (out0, out1):
        @plsc.parallel_loop(0, TPC * 16, unroll=4)
        def _(i):
            p = (i // 16) * TILE + 6 * 128 + (i % 16) * L
            obuf[pl.ds(p, L)] = zeros

    def start_idx(c, buf, sem):
        pltpu.async_copy(idx_hbm.at[pl.ds(ibase + c * CHUNK, CHUNK)], buf, sem)

    def wait_idx(c, buf, sem):
        pltpu.make_async_copy(
            idx_hbm.at[pl.ds(ibase + c * CHUNK, CHUNK)], buf, sem).wait()

    def start_out(c, buf, sem):
        pltpu.async_copy(
            buf, out_hbm.at[pl.ds(obase + c * OUT_CH, OUT_CH)], sem)

    def wait_out(c, buf, sem):
        pltpu.make_async_copy(
            buf, out_hbm.at[pl.ds(obase + c * OUT_CH, OUT_CH)], sem).wait()

    def compute(ibuf, obuf):
        @plsc.parallel_loop(0, GROUPS, unroll=8)
        def _(j):
            tv = ibuf[pl.ds(j * L, L)]
            tile_base = (j // 8) * TILE + (j % 8) * L
            for d in range(DIM):
                vals = plsc.load_gather(
                    table_v.at[pl.ds(d * 128, 128)], [tv])
                obuf[pl.ds(tile_base + d * 128, L)] = vals

    start_idx(0, idx0, sem_i0)

    def pair_body(p, _):
        c0 = p * 2
        c1 = c0 + 1
        start_idx(c1, idx1, sem_i1)
        wait_idx(c0, idx0, sem_i0)

        @pl.when(p > 0)
        def _():
            wait_out(c0 - 2, out0, sem_o0)
        compute(idx0, out0)
        start_out(c0, out0, sem_o0)

        @pl.when(p < NCHUNK // 2 - 1)
        def _():
            start_idx(c0 + 2, idx0, sem_i0)
        wait_idx(c1, idx1, sem_i1)

        @pl.when(p > 0)
        def _():
            wait_out(c1 - 2, out1, sem_o1)
        compute(idx1, out1)
        start_out(c1, out1, sem_o1)
        return 0

    lax.fori_loop(0, NCHUNK // 2, pair_body, 0)
    wait_out(NCHUNK - 2, out0, sem_o0)
    wait_out(NCHUNK - 1, out1, sem_o1)


@jax.jit
def kernel(inputs, W):
    idx_flat = inputs.reshape(-1).astype(jnp.int32)
    table_planar = jnp.zeros((8, 128), jnp.float32).at[:DIM, :VOCAB].set(W.T)

    mesh = plsc.VectorSubcoreMesh(core_axis_name="c", subcore_axis_name="s")
    out_tiles = pl.kernel(
        _sc_kernel,
        out_type=jax.ShapeDtypeStruct((NTILES * TILE,), jnp.float32),
        mesh=mesh,
        compiler_params=pltpu.CompilerParams(
            needs_layout_passes=False,
            use_tc_tiling_on_sc=False,
        ),
        scratch_types=[
            pltpu.VMEM((8 * 128,), jnp.float32),
            pltpu.VMEM((CHUNK,), jnp.int32),
            pltpu.VMEM((CHUNK,), jnp.int32),
            pltpu.VMEM((OUT_CH,), jnp.float32),
            pltpu.VMEM((OUT_CH,), jnp.float32),
            pltpu.SemaphoreType.DMA,
            pltpu.SemaphoreType.DMA,
            pltpu.SemaphoreType.DMA,
            pltpu.SemaphoreType.DMA,
        ],
    )(table_planar.reshape(-1), idx_flat)
    st = (out_tiles.reshape(NTILES, 8, 128)
          .transpose(0, 2, 1)
          .reshape(N, 8)[:, :DIM])
    return st


# speedup vs baseline: 200.8350x; 1.2103x over previous
import jax
import jax.numpy as jnp
from jax import lax
from jax.experimental import pallas as pl
from jax.experimental.pallas import tpu as pltpu
from jax.experimental.pallas import tpu_sc as plsc

T, B = 200, 16384
VOCAB, DIM = 115, 6
N = T * B
NC, NS, L = 2, 16, 16
NW = NC * NS
TR, BC = T // 8, B // 128
TILE = 1024
NTILES = N // 128
BCB = 16
NUNITS = TR * 8 * (BC // BCB)
UPW = NUNITS // NW
GROUPS = BCB * 8
OUT_CH = BCB * TILE


def _sc_kernel(table_hbm, idx_hbm, out_hbm,
               table_v, idx0, idx1, out0, out1,
               sem_i0, sem_i1, sem_o0, sem_o1):
    wid = lax.axis_index("s") * NC + lax.axis_index("c")
    ubase = wid * UPW

    pltpu.sync_copy(table_hbm, table_v)

    zeros = jnp.zeros((L,), jnp.float32)
    for obuf in (out0, out1):
        @plsc.parallel_loop(0, BCB * 16, unroll=4)
        def _(i):
            p = (i // 16) * TILE + 6 * 128 + (i % 16) * L
            obuf[pl.ds(p, L)] = zeros

    def unit_coords(c):
        u = ubase + c
        tr = u // 64
        rb = u % 64
        r = rb // 8
        bc0 = (rb % 8) * BCB
        ctile0 = (8 * tr + r) * BC + bc0
        return tr, r, bc0, ctile0

    def start_idx(c, buf, sem):
        tr, r, bc0, _ = unit_coords(c)
        pltpu.async_copy(
            idx_hbm.at[tr, pl.ds(bc0, BCB), pl.ds(r, 1), :], buf, sem)

    def wait_idx(c, buf, sem):
        tr, r, bc0, _ = unit_coords(c)
        pltpu.make_async_copy(
            idx_hbm.at[tr, pl.ds(bc0, BCB), pl.ds(r, 1), :], buf, sem).wait()

    def start_out(c, buf, sem):
        _, _, _, ctile0 = unit_coords(c)
        pltpu.async_copy(
            buf, out_hbm.at[pl.ds(ctile0 * TILE, OUT_CH)], sem)

    def wait_out(c, buf, sem):
        _, _, _, ctile0 = unit_coords(c)
        pltpu.make_async_copy(
            buf, out_hbm.at[pl.ds(ctile0 * TILE, OUT_CH)], sem).wait()

    def compute(ibuf, obuf):
        @plsc.parallel_loop(0, GROUPS, unroll=8)
        def _(j):
            tv = ibuf[j // 8, 0, pl.ds((j % 8) * L, L)]
            tile_base = (j // 8) * TILE + (j % 8) * L
            for d in range(DIM):
                vals = plsc.load_gather(
                    table_v.at[pl.ds(d * 128, 128)], [tv])
                obuf[pl.ds(tile_base + d * 128, L)] = vals

    start_idx(0, idx0, sem_i0)

    def pair_body(p, _):
        c0 = p * 2
        c1 = c0 + 1
        start_idx(c1, idx1, sem_i1)
        wait_idx(c0, idx0, sem_i0)

        @pl.when(p > 0)
        def _():
            wait_out(c0 - 2, out0, sem_o0)
        compute(idx0, out0)
        start_out(c0, out0, sem_o0)

        @pl.when(p < UPW // 2 - 1)
        def _():
            start_idx(c0 + 2, idx0, sem_i0)
        wait_idx(c1, idx1, sem_i1)

        @pl.when(p > 0)
        def _():
            wait_out(c1 - 2, out1, sem_o1)
        compute(idx1, out1)
        start_out(c1, out1, sem_o1)
        return 0

    lax.fori_loop(0, UPW // 2, pair_body, 0)
    wait_out(UPW - 2, out0, sem_o0)
    wait_out(UPW - 1, out1, sem_o1)


@jax.jit
def kernel(inputs, W):
    idx_tiles = (inputs.astype(jnp.int32)
                 .reshape(TR, 8, BC, 128)
                 .transpose(0, 2, 1, 3))
    table_planar = jnp.zeros((8, 128), jnp.float32).at[:DIM, :VOCAB].set(W.T)

    mesh = plsc.VectorSubcoreMesh(core_axis_name="c", subcore_axis_name="s")
    out_tiles = pl.kernel(
        _sc_kernel,
        out_type=jax.ShapeDtypeStruct((NTILES * TILE,), jnp.float32),
        mesh=mesh,
        compiler_params=pltpu.CompilerParams(
            needs_layout_passes=False,
            use_tc_tiling_on_sc=False,
        ),
        scratch_types=[
            pltpu.VMEM((8 * 128,), jnp.float32),
            pltpu.VMEM((BCB, 1, 128), jnp.int32),
            pltpu.VMEM((BCB, 1, 128), jnp.int32),
            pltpu.VMEM((OUT_CH,), jnp.float32),
            pltpu.VMEM((OUT_CH,), jnp.float32),
            pltpu.SemaphoreType.DMA,
            pltpu.SemaphoreType.DMA,
            pltpu.SemaphoreType.DMA,
            pltpu.SemaphoreType.DMA,
        ],
    )(table_planar.reshape(-1), idx_tiles)
    st = (out_tiles.reshape(NTILES, 8, 128)
          .transpose(0, 2, 1)
          .reshape(N, 8)[:, :DIM])
    return st


# speedup vs baseline: 212.0895x; 1.0560x over previous
import jax
import jax.numpy as jnp
from jax import lax
from jax.experimental import pallas as pl
from jax.experimental.pallas import tpu as pltpu
from jax.experimental.pallas import tpu_sc as plsc

T, B = 200, 16384
VOCAB, DIM = 115, 6
N = T * B
NC, NS, L = 2, 16, 16
NW = NC * NS
TR, BC = T // 8, B // 128
TILE = 1024
NTILES = N // 128
BCB = 16
NUNITS = TR * 8 * (BC // BCB)
UPW = NUNITS // NW
GROUPS = BCB * 8


def _sc_kernel(table_hbm, idx_hbm, out_hbm,
               table_v, idx0, idx1, out0, out1,
               sem_i0, sem_i1, sem_o0, sem_o1):
    wid = lax.axis_index("s") * NC + lax.axis_index("c")
    ubase = wid * UPW

    pltpu.sync_copy(table_hbm, table_v)

    def unit_coords(c):
        u = ubase + c
        tr = u // 64
        rb = u % 64
        r = rb // 8
        bc0 = (rb % 8) * BCB
        ctile0 = (8 * tr + r) * BC + bc0
        return tr, r, bc0, ctile0

    def start_idx(c, buf, sem):
        tr, r, bc0, _ = unit_coords(c)
        pltpu.async_copy(
            idx_hbm.at[tr, pl.ds(bc0, BCB), pl.ds(r, 1), :], buf, sem)

    def wait_idx(c, buf, sem):
        tr, r, bc0, _ = unit_coords(c)
        pltpu.make_async_copy(
            idx_hbm.at[tr, pl.ds(bc0, BCB), pl.ds(r, 1), :], buf, sem).wait()

    def start_out(c, buf, sem):
        _, _, _, ctile0 = unit_coords(c)
        pltpu.async_copy(
            buf, out_hbm.at[pl.ds(ctile0, BCB), pl.ds(0, DIM), :], sem)

    def wait_out(c, buf, sem):
        _, _, _, ctile0 = unit_coords(c)
        pltpu.make_async_copy(
            buf, out_hbm.at[pl.ds(ctile0, BCB), pl.ds(0, DIM), :], sem).wait()

    def compute(ibuf, obuf):
        @plsc.parallel_loop(0, GROUPS, unroll=8)
        def _(j):
            tv = ibuf[j // 8, 0, pl.ds((j % 8) * L, L)]
            for d in range(DIM):
                vals = plsc.load_gather(
                    table_v.at[pl.ds(d * 128, 128)], [tv])
                obuf[j // 8, d, pl.ds((j % 8) * L, L)] = vals

    start_idx(0, idx0, sem_i0)

    def pair_body(p, _):
        c0 = p * 2
        c1 = c0 + 1
        start_idx(c1, idx1, sem_i1)
        wait_idx(c0, idx0, sem_i0)

        @pl.when(p > 0)
        def _():
            wait_out(c0 - 2, out0, sem_o0)
        compute(idx0, out0)
        start_out(c0, out0, sem_o0)

        @pl.when(p < UPW // 2 - 1)
        def _():
            start_idx(c0 + 2, idx0, sem_i0)
        wait_idx(c1, idx1, sem_i1)

        @pl.when(p > 0)
        def _():
            wait_out(c1 - 2, out1, sem_o1)
        compute(idx1, out1)
        start_out(c1, out1, sem_o1)
        return 0

    lax.fori_loop(0, UPW // 2, pair_body, 0)
    wait_out(UPW - 2, out0, sem_o0)
    wait_out(UPW - 1, out1, sem_o1)


@jax.jit
def kernel(inputs, W):
    idx_tiles = (inputs.astype(jnp.int32)
                 .reshape(TR, 8, BC, 128)
                 .transpose(0, 2, 1, 3))
    table_planar = jnp.zeros((8, 128), jnp.float32).at[:DIM, :VOCAB].set(W.T)

    mesh = plsc.VectorSubcoreMesh(core_axis_name="c", subcore_axis_name="s")
    out_tiles = pl.kernel(
        _sc_kernel,
        out_type=jax.ShapeDtypeStruct((NTILES, 8, 128), jnp.float32),
        mesh=mesh,
        compiler_params=pltpu.CompilerParams(
            needs_layout_passes=False,
            use_tc_tiling_on_sc=False,
        ),
        scratch_types=[
            pltpu.VMEM((8 * 128,), jnp.float32),
            pltpu.VMEM((BCB, 1, 128), jnp.int32),
            pltpu.VMEM((BCB, 1, 128), jnp.int32),
            pltpu.VMEM((BCB, DIM, 128), jnp.float32),
            pltpu.VMEM((BCB, DIM, 128), jnp.float32),
            pltpu.SemaphoreType.DMA,
            pltpu.SemaphoreType.DMA,
            pltpu.SemaphoreType.DMA,
            pltpu.SemaphoreType.DMA,
        ],
    )(table_planar.reshape(-1), idx_tiles)
    st = out_tiles.transpose(0, 2, 1).reshape(N, 8)[:, :DIM]
    return st
